# Initial kernel scaffold; baseline (speedup 1.0000x reference)
import functools

import jax
import jax.numpy as jnp
from jax import lax
from jax.experimental import pallas as pl
from jax.experimental.pallas import tpu as pltpu
from jax.experimental.pallas import tpu_sc as plsc


def kernel(words, w_table, w_bias):
    B = words.shape[0]
    V, D = w_table.shape
    info = plsc.get_sparse_core_info()
    NC, NS = info.num_cores, info.num_subcores
    NW = NC * NS
    b_per_w = B // NW
    mesh = plsc.VectorSubcoreMesh(core_axis_name="c", subcore_axis_name="s")

    @functools.partial(
        pl.kernel,
        mesh=mesh,
        out_type=(
            jax.ShapeDtypeStruct((B, D), jnp.float32),
            jax.ShapeDtypeStruct((B, 1), jnp.float32),
        ),
        scratch_types=[
            pltpu.VMEM((b_per_w,), jnp.int32),
            pltpu.VMEM((b_per_w, D), jnp.float32),
            pltpu.VMEM((b_per_w, 1), jnp.float32),
            pltpu.SemaphoreType.DMA,
            pltpu.SemaphoreType.DMA,
        ],
    )
    def glove_gather(words_hbm, table_hbm, bias_hbm, emb_hbm, bout_hbm,
                     idx_v, rows_v, bias_v, sem_rows, sem_bias):
        wid = lax.axis_index("s") * NC + lax.axis_index("c")
        base = wid * b_per_w
        pltpu.sync_copy(words_hbm.at[pl.ds(base, b_per_w)], idx_v)
        c_rows = pltpu.async_copy(table_hbm.at[idx_v], rows_v, sem_rows)
        c_bias = pltpu.async_copy(bias_hbm.at[idx_v], bias_v, sem_bias)
        c_rows.wait()
        pltpu.sync_copy(rows_v, emb_hbm.at[pl.ds(base, b_per_w)])
        c_bias.wait()
        pltpu.sync_copy(bias_v, bout_hbm.at[pl.ds(base, b_per_w)])

    return glove_gather(words, w_table, w_bias)


# speedup vs baseline: 1.3218x; 1.3218x over previous
import functools

import jax
import jax.numpy as jnp
from jax import lax
from jax.experimental import pallas as pl
from jax.experimental.pallas import tpu as pltpu
from jax.experimental.pallas import tpu_sc as plsc


def kernel(words, w_table, w_bias):
    B = words.shape[0]
    V, D = w_table.shape
    info = plsc.get_sparse_core_info()
    NC, NS = info.num_cores, info.num_subcores
    NW = NC * NS
    b_per_w = B // NW
    mesh = plsc.VectorSubcoreMesh(core_axis_name="c", subcore_axis_name="s")

    @functools.partial(
        pl.kernel,
        mesh=mesh,
        out_type=(
            jax.ShapeDtypeStruct((B, D), jnp.float32),
            jax.ShapeDtypeStruct((B,), jnp.float32),
        ),
        scratch_types=[
            pltpu.VMEM((b_per_w,), jnp.int32),
            pltpu.VMEM((b_per_w, D), jnp.float32),
            pltpu.VMEM((b_per_w,), jnp.float32),
            pltpu.SemaphoreType.DMA,
            pltpu.SemaphoreType.DMA,
        ],
    )
    def glove_gather(words_hbm, table_hbm, bias_hbm, emb_hbm, bout_hbm,
                     idx_v, rows_v, bias_v, sem_rows, sem_bias):
        wid = lax.axis_index("s") * NC + lax.axis_index("c")
        base = wid * b_per_w
        pltpu.sync_copy(words_hbm.at[pl.ds(base, b_per_w)], idx_v)
        c_rows = pltpu.async_copy(table_hbm.at[idx_v], rows_v, sem_rows)
        c_bias = pltpu.async_copy(bias_hbm.at[idx_v], bias_v, sem_bias)
        c_rows.wait()
        pltpu.sync_copy(rows_v, emb_hbm.at[pl.ds(base, b_per_w)])
        c_bias.wait()
        pltpu.sync_copy(bias_v, bout_hbm.at[pl.ds(base, b_per_w)])

    emb, bias = glove_gather(words, w_table, w_bias.reshape(V))
    return emb, bias.reshape(B, 1)
